# Initial kernel scaffold; baseline (speedup 1.0000x reference)
import jax
import jax.numpy as jnp
from jax.experimental import pallas as pl


def kernel(loc_data, conf_data, landm_data, priors, targets):
    raise NotImplementedError("write your pallas kernel here")


# speedup vs baseline: 46.5481x; 46.5481x over previous
import jax
import jax.numpy as jnp
from jax.experimental import pallas as pl
from jax.experimental.pallas import tpu as pltpu

_P = 16800
_ROWS = 132
_PPAD = _ROWS * 128
_NOBJ = 16
_TH = 0.35
_NEGPOS = 7
_V0 = 0.1
_V1 = 0.2


def _mbl_kernel(t_ref, pr_ref, loc_ref, conf_ref, lm_ref, out_ref):
    b = pl.program_id(0)
    f32 = jnp.float32

    pcx = pr_ref[0]
    pcy = pr_ref[1]
    pw = pr_ref[2]
    ph = pr_ref[3]
    px0 = pcx - pw / 2.0
    py0 = pcy - ph / 2.0
    px1 = pcx + pw / 2.0
    py1 = pcy + ph / 2.0
    area_p = (px1 - px0) * (py1 - py0)

    ri = jax.lax.broadcasted_iota(jnp.int32, (_ROWS, 128), 0)
    ci = jax.lax.broadcasted_iota(jnp.int32, (_ROWS, 128), 1)
    fi = ri * 128 + ci
    valid = fi < _P

    bto = jnp.full((_ROWS, 128), -2.0, f32)
    bti = jnp.zeros((_ROWS, 128), jnp.int32)
    bpis = []
    for i in range(_NOBJ):
        tx0 = t_ref[0, i, 0]
        ty0 = t_ref[0, i, 1]
        tx1 = t_ref[0, i, 2]
        ty1 = t_ref[0, i, 3]
        iw = jnp.maximum(jnp.minimum(px1, tx1) - jnp.maximum(px0, tx0), 0.0)
        ih = jnp.maximum(jnp.minimum(py1, ty1) - jnp.maximum(py0, ty0), 0.0)
        inter = iw * ih
        area_t = (tx1 - tx0) * (ty1 - ty0)
        ov = inter / (area_t + area_p - inter)
        ov = jnp.where(valid, ov, -1.0)
        upd = ov > bto
        bto = jnp.where(upd, ov, bto)
        bti = jnp.where(upd, i, bti)
        m = jnp.max(ov)
        bpis.append(jnp.min(jnp.where(ov == m, fi, _PPAD)))
    for i in range(_NOBJ):
        eq = fi == bpis[i]
        bto = jnp.where(eq, 2.0, bto)
        bti = jnp.where(eq, i, bti)

    pos = bto >= _TH
    pm = pos.astype(f32)
    npos = jnp.sum(pm)

    zero = jnp.zeros((_ROWS, 128), f32)
    masks = [(bti == i).astype(f32) for i in range(_NOBJ)]
    mt = []
    for c in range(14):
        acc = zero
        for i in range(_NOBJ):
            acc = acc + masks[i] * t_ref[0, i, c]
        mt.append(acc)

    g = [
        ((mt[0] + mt[2]) / 2.0 - pcx) / (_V0 * pw),
        ((mt[1] + mt[3]) / 2.0 - pcy) / (_V0 * ph),
        jnp.log((mt[2] - mt[0]) / pw) / _V1,
        jnp.log((mt[3] - mt[1]) / ph) / _V1,
    ]
    ll = zero
    for c in range(4):
        d = jnp.abs(loc_ref[0, c] - g[c])
        ll = ll + jnp.where(d < 1.0, 0.5 * d * d, d - 0.5)
    loss_l = jnp.sum(ll * pm)

    llm = zero
    for k in range(5):
        gx = (mt[4 + 2 * k] - pcx) / (_V0 * pw)
        gy = (mt[5 + 2 * k] - pcy) / (_V0 * ph)
        dx = jnp.abs(lm_ref[0, 2 * k] - gx)
        dy = jnp.abs(lm_ref[0, 2 * k + 1] - gy)
        llm = llm + jnp.where(dx < 1.0, 0.5 * dx * dx, dx - 0.5)
        llm = llm + jnp.where(dy < 1.0, 0.5 * dy * dy, dy - 0.5)
    loss_lm = jnp.sum(llm * pm)

    c0 = conf_ref[0, 0]
    c1 = conf_ref[0, 1]
    mx = jnp.maximum(c0, c1)
    lse = mx + jnp.log(jnp.exp(c0 - mx) + jnp.exp(c1 - mx))
    gath = jnp.where(pos, c1, c0)
    ce = jnp.where(valid, lse - gath, 0.0)

    ceb = jax.lax.bitcast_convert_type(ce, jnp.int32)
    k = jnp.minimum(_NEGPOS * npos.astype(jnp.int32), _P - 1)

    def body(_, lohi):
        lo, hi = lohi
        mid = lo + (hi - lo + 1) // 2
        cnt = jnp.sum(jnp.where(ceb >= mid, 1, 0))
        big = cnt >= k
        return jnp.where(big, mid, lo), jnp.where(big, hi, mid - 1)

    lo, _hi = jax.lax.fori_loop(
        0, 31, body, (jnp.int32(0), jnp.max(ceb)))
    sel = jnp.logical_or(pos, ceb >= lo)
    loss_c = jnp.sum(ce * sel.astype(f32))

    lane = jax.lax.broadcasted_iota(jnp.int32, (1, 128), 1)
    vec = (jnp.where(lane == 0, loss_l, 0.0)
           + jnp.where(lane == 1, loss_c, 0.0)
           + jnp.where(lane == 2, loss_lm, 0.0)
           + jnp.where(lane == 3, npos, 0.0))

    @pl.when(b == 0)
    def _():
        out_ref[...] = jnp.zeros_like(out_ref)

    out_ref[...] += vec


def _planar(x):
    b, p, c = x.shape
    xt = jnp.moveaxis(x, 2, 1)
    xt = jnp.pad(xt, ((0, 0), (0, 0), (0, _PPAD - p)))
    return xt.reshape(b, c, _ROWS, 128)


def kernel(loc_data, conf_data, landm_data, priors, targets):
    num = loc_data.shape[0]
    locp = _planar(loc_data)
    confp = _planar(conf_data)
    lmp = _planar(landm_data)
    prp = _planar(priors[None])[0]

    out = pl.pallas_call(
        _mbl_kernel,
        grid=(num,),
        in_specs=[
            pl.BlockSpec((1, _NOBJ, 15), lambda b: (b, 0, 0)),
            pl.BlockSpec((4, _ROWS, 128), lambda b: (0, 0, 0)),
            pl.BlockSpec((1, 4, _ROWS, 128), lambda b: (b, 0, 0, 0)),
            pl.BlockSpec((1, 2, _ROWS, 128), lambda b: (b, 0, 0, 0)),
            pl.BlockSpec((1, 10, _ROWS, 128), lambda b: (b, 0, 0, 0)),
        ],
        out_specs=pl.BlockSpec((1, 128), lambda b: (0, 0)),
        out_shape=jax.ShapeDtypeStruct((1, 128), jnp.float32),
        compiler_params=pltpu.CompilerParams(
            dimension_semantics=("arbitrary",)),
    )(targets, prp, locp, confp, lmp)

    s = out[0]
    n = jnp.maximum(s[3], 1.0)
    return (s[0] / n, s[1] / n, s[2] / n)
